# Initial kernel scaffold; baseline (speedup 1.0000x reference)
import jax
import jax.numpy as jnp
from jax.experimental import pallas as pl


def kernel(input, rel, edge_index, rel_idx, adj, w_rel, bias):
    raise NotImplementedError("write your pallas kernel here")


# speedup vs baseline: 3.8232x; 3.8232x over previous
import functools

import jax
import jax.numpy as jnp
from jax import lax
from jax.experimental import pallas as pl
from jax.experimental.pallas import tpu as pltpu
from jax.experimental.pallas import tpu_sc as plsc

N = 4096
D = 256
R_PAD = 1024
REP = 8
E = 65536


def _scores_body(relt_ref, w_ref, out_ref):
    s = jnp.sum(relt_ref[...] * w_ref[...], axis=0, keepdims=True)
    out_ref[...] = jnp.broadcast_to(jnp.maximum(s, 0.0), (REP, R_PAD))


def _compute_scores(relt_pad, w_col):
    return pl.pallas_call(
        _scores_body,
        out_shape=jax.ShapeDtypeStruct((REP, R_PAD), jnp.float32),
    )(relt_pad, w_col)


_NC = 2
_NS = 16
_NW = _NC * _NS
_EPW = E // _NW
_CH = 128
_NCHUNK = _EPW // _CH

_sc_mesh = plsc.VectorSubcoreMesh(core_axis_name="c", subcore_axis_name="s")


@functools.partial(
    pl.kernel,
    mesh=_sc_mesh,
    out_type=(),
    scratch_types=[
        pltpu.VMEM((_NCHUNK, _CH), jnp.int32),
        pltpu.VMEM((_NCHUNK, _CH), jnp.int32),
        pltpu.VMEM((_NCHUNK, _CH), jnp.int32),
        pltpu.VMEM((_NCHUNK, _CH), jnp.int32),
        pltpu.VMEM((_NCHUNK, _CH), jnp.int32),
        pltpu.VMEM((_NCHUNK, _CH), jnp.float32),
        pltpu.SemaphoreType.DMA,
    ],
)
def _sc_scatter(rows_hbm, cols_hbm, ridx_hbm, table_hbm, l_ref,
                rows_v, cols_v, ridx_v, gidx_v, flat_v, val_v, sem):
    wid = lax.axis_index("s") * _NC + lax.axis_index("c")
    rbase = wid * _NCHUNK
    pltpu.sync_copy(rows_hbm.at[pl.ds(rbase, _NCHUNK)], rows_v)
    pltpu.sync_copy(cols_hbm.at[pl.ds(rbase, _NCHUNK)], cols_v)
    pltpu.sync_copy(ridx_hbm.at[pl.ds(rbase, _NCHUNK)], ridx_v)

    spread = (lax.iota(jnp.int32, 16) & (REP - 1)) * R_PAD
    for c in range(_NCHUNK):
        @pl.loop(0, _CH // 16)
        def _chunk_body(j, c=c):
            sl = pl.ds(j * 16, 16)
            gidx_v[c, sl] = ridx_v[c, sl] + spread
            flat_v[c, sl] = rows_v[c, sl] * N + cols_v[c, sl]

    gathers = [
        pltpu.async_copy(table_hbm.at[gidx_v.at[c]], val_v.at[c], sem)
        for c in range(_NCHUNK)
    ]
    for cp in gathers:
        cp.wait()
    scatters = [
        pltpu.async_copy(val_v.at[c], l_ref.at[flat_v.at[c]], sem)
        for c in range(_NCHUNK)
    ]
    for cp in scatters:
        cp.wait()


_BR = 256


def _flash_body(l_ref, adj_ref, inp_ref, bias_ref, out_ref):
    m = l_ref[...] + adj_ref[...]
    mx = jnp.max(m, axis=1, keepdims=True)
    e = jnp.exp(m - mx)
    z = jnp.sum(e, axis=1, keepdims=True)
    acc = jnp.dot(e, inp_ref[...], preferred_element_type=jnp.float32)
    r = acc / z + bias_ref[...]
    out_ref[...] = jnp.where(r > 0.0, r, jnp.exp(r) - 1.0)


def _flash_call(lmat, adj, inp, bias_row):
    grid = (N // _BR,)
    return pl.pallas_call(
        _flash_body,
        grid=grid,
        in_specs=[
            pl.BlockSpec((_BR, N), lambda i: (i, 0)),
            pl.BlockSpec((_BR, N), lambda i: (i, 0)),
            pl.BlockSpec((N, D), lambda i: (0, 0)),
            pl.BlockSpec((1, D), lambda i: (0, 0)),
        ],
        out_specs=pl.BlockSpec((_BR, D), lambda i: (i, 0)),
        out_shape=jax.ShapeDtypeStruct((N, D), jnp.float32),
    )(lmat, adj, inp, bias_row)


def kernel(input, rel, edge_index, rel_idx, adj, w_rel, bias):
    relt_pad = jnp.pad(rel, ((0, R_PAD - rel.shape[0]), (0, 0))).T
    table = _compute_scores(relt_pad, w_rel.reshape(-1, 1)).reshape(REP * R_PAD)
    e1 = edge_index[0].reshape(E // _CH, _CH)
    e2 = edge_index[1].reshape(E // _CH, _CH)
    ridx2 = rel_idx.reshape(E // _CH, _CH)
    l_ref = jax.new_ref(jnp.zeros((N * N,), jnp.float32))
    _sc_scatter(e1, e2, ridx2, table, l_ref)
    _sc_scatter(e2, e1, ridx2, table, l_ref)
    lmat = l_ref[...].reshape(N, N)
    return _flash_call(lmat, adj, input, bias.reshape(1, D))
